# Initial kernel scaffold; baseline (speedup 1.0000x reference)
import functools

import jax
import jax.numpy as jnp
from jax import lax
from jax.experimental import pallas as pl
from jax.experimental.pallas import tpu as pltpu
from jax.experimental.pallas import tpu_sc as plsc

DIM = 96
QK = 96
NWIN = 7
HEADS = 8
TOPK = 4
RATIO = 4
SCALE = QK ** -0.5
WS = 32
P2 = NWIN * NWIN
W2K = (WS // RATIO) ** 2
HD = QK // HEADS
CKV = QK + DIM
ROWS = 2 * P2


def _proj_pool_body(x_ref, wv_ref, bv_ref, wkv_ref, bkv_ref,
                    v_ref, kvp_ref, xm_ref):
    xw = x_ref[0].reshape(WS * WS, DIM)
    v = jnp.dot(xw, wv_ref[0], preferred_element_type=jnp.float32) + bv_ref[0]
    v_ref[0] = v.reshape(WS, WS, DIM)
    t = x_ref[0].reshape(WS, WS // RATIO, RATIO, DIM)
    t = t[:, :, 0] + t[:, :, 1] + t[:, :, 2] + t[:, :, 3]
    t = t.reshape(WS // RATIO, RATIO, WS // RATIO, DIM)
    t = t[:, 0] + t[:, 1] + t[:, 2] + t[:, 3]
    pooled = t.reshape(W2K, DIM) * (1.0 / (RATIO * RATIO))
    kvp_ref[0, 0] = (jnp.dot(pooled, wkv_ref[0],
                             preferred_element_type=jnp.float32) + bkv_ref[0])
    xm_ref[0, 0] = jnp.mean(xw, axis=0).reshape(1, DIM)


def _run_proj_pool(x2, Wv, bv, Wkv, bkv):
    return pl.pallas_call(
        _proj_pool_body,
        grid=(2, NWIN, NWIN),
        in_specs=[
            pl.BlockSpec((1, WS, WS, DIM), lambda s, j, i: (s, j, i, 0)),
            pl.BlockSpec((1, DIM, DIM), lambda s, j, i: (s, 0, 0)),
            pl.BlockSpec((1, 1, DIM), lambda s, j, i: (s, 0, 0)),
            pl.BlockSpec((1, DIM, CKV), lambda s, j, i: (s, 0, 0)),
            pl.BlockSpec((1, 1, CKV), lambda s, j, i: (s, 0, 0)),
        ],
        out_specs=[
            pl.BlockSpec((1, WS, WS, DIM), lambda s, j, i: (s, j, i, 0)),
            pl.BlockSpec((1, 1, W2K, CKV), lambda s, j, i: (s, j * NWIN + i, 0, 0)),
            pl.BlockSpec((1, 1, 1, DIM), lambda s, j, i: (s, j * NWIN + i, 0, 0)),
        ],
        out_shape=[
            jax.ShapeDtypeStruct((2, 224, 224, DIM), jnp.float32),
            jax.ShapeDtypeStruct((2, P2, W2K, CKV), jnp.float32),
            jax.ShapeDtypeStruct((2, P2, 1, DIM), jnp.float32),
        ],
    )(x2, Wv, bv, Wkv, bkv)


def _router_logits_body(xm_ref, wq_ref, bq_ref, wk_ref, bk_ref, out_ref):
    xm = xm_ref[0]
    qw = jnp.dot(xm, wq_ref[0], preferred_element_type=jnp.float32) + bq_ref[0]
    kw = jnp.dot(xm, wk_ref[0], preferred_element_type=jnp.float32) + bk_ref[0]
    kw_pad = jnp.concatenate([kw, jnp.zeros((64 - P2, QK), jnp.float32)], axis=0)
    lg = lax.dot_general(qw * SCALE, kw_pad, (((1,), (1,)), ((), ())),
                         preferred_element_type=jnp.float32)
    col = lax.broadcasted_iota(jnp.int32, (P2, 64), 1)
    out_ref[0] = jnp.where(col < P2, lg, -1e30)


def _run_router_logits(xm2, Wq, bq, Wk, bk):
    return pl.pallas_call(
        _router_logits_body,
        grid=(2,),
        in_specs=[
            pl.BlockSpec((1, P2, DIM), lambda s: (s, 0, 0)),
            pl.BlockSpec((1, DIM, QK), lambda s: (s, 0, 0)),
            pl.BlockSpec((1, 1, QK), lambda s: (s, 0, 0)),
            pl.BlockSpec((1, DIM, QK), lambda s: (s, 0, 0)),
            pl.BlockSpec((1, 1, QK), lambda s: (s, 0, 0)),
        ],
        out_specs=pl.BlockSpec((1, P2, 64), lambda s: (s, 0, 0)),
        out_shape=jax.ShapeDtypeStruct((2, P2, 64), jnp.float32),
    )(xm2, Wq, bq, Wk, bk)


def _topk98(logits2):
    info = plsc.get_sparse_core_info()
    nc, ns = info.num_cores, info.num_subcores
    workers = nc * ns
    rpw = (ROWS + workers - 1) // workers
    mesh = plsc.VectorSubcoreMesh(core_axis_name="c", subcore_axis_name="s")

    @functools.partial(
        pl.kernel,
        mesh=mesh,
        out_type=jax.ShapeDtypeStruct((ROWS, 8), jnp.int32),
        scratch_types=[
            pltpu.VMEM((64,), jnp.float32),
            pltpu.VMEM((64,), jnp.float32),
            pltpu.VMEM((64,), jnp.int32),
            pltpu.VMEM((16,), jnp.int32),
        ],
    )
    def run(lg_hbm, out_hbm, lrow, skeys, svals, fvals):
        wid = lax.axis_index("s") * nc + lax.axis_index("c")
        for r in range(rpw):
            row = wid * rpw + r

            @pl.when(row < ROWS)
            def _():
                pltpu.sync_copy(lg_hbm.at[row], lrow)
                for c in range(4):
                    keys = lrow[pl.ds(16 * c, 16)]
                    vals = lax.iota(jnp.int32, (16,)) + 16 * c
                    sk, sv = plsc.sort_key_val(keys, vals, descending=True)
                    skeys[pl.ds(16 * c, 16)] = sk
                    svals[pl.ds(16 * c, 16)] = sv
                lane = lax.iota(jnp.int32, (16,))
                cand = (lane // 4) * 16 + (lane % 4)
                ck = plsc.load_gather(skeys, [cand])
                cv = plsc.load_gather(svals, [cand])
                _, fv = plsc.sort_key_val(ck, cv, descending=True)
                fvals[...] = fv
                pltpu.sync_copy(fvals.at[pl.ds(0, 8)], out_hbm.at[row])

    return run(logits2)


def _attn_body(idx_ref, x_ref, wq_ref, bq_ref, kvp_ref, o_ref):
    s = pl.program_id(0)
    w = pl.program_id(1)
    xw = x_ref[0].reshape(WS * WS, DIM)
    q = jnp.dot(xw, wq_ref[0], preferred_element_type=jnp.float32) + bq_ref[0]
    kv = jnp.concatenate(
        [kvp_ref[0, idx_ref[s, w, t]] for t in range(TOPK)], axis=0)
    k = kv[:, :QK]
    v = kv[:, QK:]
    col = lax.broadcasted_iota(jnp.int32, (1, QK), 1)
    o = jnp.zeros((WS * WS, DIM), jnp.float32)
    for h in range(HEADS):
        m = (col >= HD * h) & (col < HD * (h + 1))
        kh = jnp.where(m, k, 0.0)
        sh = lax.dot_general(q, kh, (((1,), (1,)), ((), ())),
                             preferred_element_type=jnp.float32) * SCALE
        sh = sh - jnp.max(sh, axis=1, keepdims=True)
        p = jnp.exp(sh)
        p = p / jnp.sum(p, axis=1, keepdims=True)
        vh = jnp.where(m, v, 0.0)
        o = o + jnp.dot(p, vh, preferred_element_type=jnp.float32)
    o_ref[0] = o.reshape(WS, WS, DIM)


def _run_attention(idx, x2, Wq, bq, kvp):
    grid_spec = pltpu.PrefetchScalarGridSpec(
        num_scalar_prefetch=1,
        grid=(2, P2),
        in_specs=[
            pl.BlockSpec((1, WS, WS, DIM),
                         lambda s, w, idx: (s, w // NWIN, w % NWIN, 0)),
            pl.BlockSpec((1, DIM, QK), lambda s, w, idx: (s, 0, 0)),
            pl.BlockSpec((1, 1, QK), lambda s, w, idx: (s, 0, 0)),
            pl.BlockSpec((1, P2, W2K, CKV), lambda s, w, idx: (s, 0, 0, 0)),
        ],
        out_specs=pl.BlockSpec((1, WS, WS, DIM),
                               lambda s, w, idx: (s, w // NWIN, w % NWIN, 0)),
    )
    return pl.pallas_call(
        _attn_body,
        grid_spec=grid_spec,
        out_shape=jax.ShapeDtypeStruct((2, 224, 224, DIM), jnp.float32),
    )(idx, x2, Wq, bq, kvp)


def _lepe_proj_body(attn_ref, vp_ref, vc_ref, vn_ref, w9_ref, bl_ref,
                    wo_ref, bo_ref, o_ref):
    j = pl.program_id(1)
    top = jnp.where(j == 0, 0.0, vp_ref[0, WS - 1:WS])
    bot = jnp.where(j == NWIN - 1, 0.0, vn_ref[0, 0:1])
    ext = jnp.concatenate([top, vc_ref[0], bot], axis=0)
    acc = jnp.zeros((WS, 224, DIM), jnp.float32)
    zcol = jnp.zeros((WS, 1, DIM), jnp.float32)
    for a in range(3):
        rows = ext[a:a + WS]
        for b in range(3):
            if b == 0:
                sh = jnp.concatenate([zcol, rows[:, :223]], axis=1)
            elif b == 2:
                sh = jnp.concatenate([rows[:, 1:], zcol], axis=1)
            else:
                sh = rows
            acc = acc + sh * w9_ref[0, 3 * a + b]
    tot = attn_ref[0] + acc + bl_ref[0, 0]
    flat = tot.reshape(WS * 224, DIM)
    out = jnp.dot(flat, wo_ref[...], preferred_element_type=jnp.float32) + bo_ref[...]
    o_ref[0] = out.reshape(WS, 224, DIM)


def _run_lepe_proj(attn, vimg, w9, bl, W_wo, b_wo):
    return pl.pallas_call(
        _lepe_proj_body,
        grid=(2, NWIN),
        in_specs=[
            pl.BlockSpec((1, WS, 224, DIM), lambda s, j: (s, j, 0, 0)),
            pl.BlockSpec((1, WS, 224, DIM),
                         lambda s, j: (s, jnp.maximum(j - 1, 0), 0, 0)),
            pl.BlockSpec((1, WS, 224, DIM), lambda s, j: (s, j, 0, 0)),
            pl.BlockSpec((1, WS, 224, DIM),
                         lambda s, j: (s, jnp.minimum(j + 1, NWIN - 1), 0, 0)),
            pl.BlockSpec((1, 9, DIM), lambda s, j: (s, 0, 0)),
            pl.BlockSpec((1, 1, DIM), lambda s, j: (s, 0, 0)),
            pl.BlockSpec((DIM, DIM), lambda s, j: (0, 0)),
            pl.BlockSpec((1, DIM), lambda s, j: (0, 0)),
        ],
        out_specs=pl.BlockSpec((1, WS, 224, DIM), lambda s, j: (s, j, 0, 0)),
        out_shape=jax.ShapeDtypeStruct((2, 224, 224, DIM), jnp.float32),
    )(attn, vimg, vimg, vimg, w9, bl, W_wo, b_wo)


def kernel(x, y, W_qkv_x, b_qkv_x, W_qkv_y, b_qkv_y,
           lepe_x_w, lepe_x_b, lepe_y_w, lepe_y_b, W_wo, b_wo):
    x2 = jnp.concatenate([x, y], axis=0)
    Wq = jnp.stack([W_qkv_x[:, :QK], W_qkv_y[:, :QK]])
    Wk = jnp.stack([W_qkv_x[:, QK:2 * QK], W_qkv_y[:, QK:2 * QK]])
    Wv = jnp.stack([W_qkv_x[:, 2 * QK:], W_qkv_y[:, 2 * QK:]])
    Wkv = jnp.stack([W_qkv_x[:, QK:], W_qkv_y[:, QK:]])
    bq = jnp.stack([b_qkv_x[:QK], b_qkv_y[:QK]]).reshape(2, 1, QK)
    bk = jnp.stack([b_qkv_x[QK:2 * QK], b_qkv_y[QK:2 * QK]]).reshape(2, 1, QK)
    bv = jnp.stack([b_qkv_x[2 * QK:], b_qkv_y[2 * QK:]]).reshape(2, 1, DIM)
    bkv = jnp.stack([b_qkv_x[QK:], b_qkv_y[QK:]]).reshape(2, 1, CKV)
    w9 = jnp.stack([lepe_x_w.reshape(DIM, 9).T, lepe_y_w.reshape(DIM, 9).T])
    bl = jnp.stack([lepe_x_b, lepe_y_b]).reshape(2, 1, DIM)

    vimg, kvp, xm = _run_proj_pool(x2, Wv, bv, Wkv, bkv)
    logits = _run_router_logits(xm.reshape(2, P2, DIM), Wq, bq, Wk, bk)
    idx = _topk98(logits.reshape(ROWS, 64))
    attn = _run_attention(idx.reshape(2, P2, 8), x2, Wq, bq, kvp)
    out = _run_lepe_proj(attn, vimg, w9, bl, W_wo, b_wo.reshape(1, DIM))
    return (out[0:1], out[1:2])


# speedup vs baseline: 1.2077x; 1.2077x over previous
import functools

import jax
import jax.numpy as jnp
from jax import lax
from jax.experimental import pallas as pl
from jax.experimental.pallas import tpu as pltpu
from jax.experimental.pallas import tpu_sc as plsc

DIM = 96
QK = 96
NWIN = 7
HEADS = 8
TOPK = 4
RATIO = 4
SCALE = QK ** -0.5
WS = 32
P2 = NWIN * NWIN
W2K = (WS // RATIO) ** 2
HD = QK // HEADS
CKV = QK + DIM
ROWS = 2 * P2


def _proj_pool_body(x_ref, wv_ref, bv_ref, wkv_ref, bkv_ref,
                    v_ref, kvp_ref, xm_ref):
    xw = x_ref[0].reshape(WS * WS, DIM)
    v = jnp.dot(xw, wv_ref[0], preferred_element_type=jnp.float32, precision=lax.Precision.HIGHEST) + bv_ref[0]
    v_ref[0] = v.reshape(WS, WS, DIM)
    t = x_ref[0].reshape(WS, WS // RATIO, RATIO, DIM)
    t = t[:, :, 0] + t[:, :, 1] + t[:, :, 2] + t[:, :, 3]
    t = t.reshape(WS // RATIO, RATIO, WS // RATIO, DIM)
    t = t[:, 0] + t[:, 1] + t[:, 2] + t[:, 3]
    pooled = t.reshape(W2K, DIM) * (1.0 / (RATIO * RATIO))
    kvp_ref[0, 0] = (jnp.dot(pooled, wkv_ref[0],
                             preferred_element_type=jnp.float32, precision=lax.Precision.HIGHEST) + bkv_ref[0])
    xm_ref[0, 0] = jnp.mean(xw, axis=0).reshape(1, DIM)


def _run_proj_pool(x2, Wv, bv, Wkv, bkv):
    return pl.pallas_call(
        _proj_pool_body,
        grid=(2, NWIN, NWIN),
        in_specs=[
            pl.BlockSpec((1, WS, WS, DIM), lambda s, j, i: (s, j, i, 0)),
            pl.BlockSpec((1, DIM, DIM), lambda s, j, i: (s, 0, 0)),
            pl.BlockSpec((1, 1, DIM), lambda s, j, i: (s, 0, 0)),
            pl.BlockSpec((1, DIM, CKV), lambda s, j, i: (s, 0, 0)),
            pl.BlockSpec((1, 1, CKV), lambda s, j, i: (s, 0, 0)),
        ],
        out_specs=[
            pl.BlockSpec((1, WS, WS, DIM), lambda s, j, i: (s, j, i, 0)),
            pl.BlockSpec((1, 1, W2K, CKV), lambda s, j, i: (s, j * NWIN + i, 0, 0)),
            pl.BlockSpec((1, 1, 1, DIM), lambda s, j, i: (s, j * NWIN + i, 0, 0)),
        ],
        out_shape=[
            jax.ShapeDtypeStruct((2, 224, 224, DIM), jnp.float32),
            jax.ShapeDtypeStruct((2, P2, W2K, CKV), jnp.float32),
            jax.ShapeDtypeStruct((2, P2, 1, DIM), jnp.float32),
        ],
    )(x2, Wv, bv, Wkv, bkv)


def _router_logits_body(xm_ref, wq_ref, bq_ref, wk_ref, bk_ref, out_ref):
    xm = xm_ref[0]
    qw = jnp.dot(xm, wq_ref[0], preferred_element_type=jnp.float32, precision=lax.Precision.HIGHEST) + bq_ref[0]
    kw = jnp.dot(xm, wk_ref[0], preferred_element_type=jnp.float32, precision=lax.Precision.HIGHEST) + bk_ref[0]
    kw_pad = jnp.concatenate([kw, jnp.zeros((64 - P2, QK), jnp.float32)], axis=0)
    lg = lax.dot_general(qw * SCALE, kw_pad, (((1,), (1,)), ((), ())),
                         preferred_element_type=jnp.float32, precision=lax.Precision.HIGHEST)
    col = lax.broadcasted_iota(jnp.int32, (P2, 64), 1)
    out_ref[0] = jnp.where(col < P2, lg, -1e30)


def _run_router_logits(xm2, Wq, bq, Wk, bk):
    return pl.pallas_call(
        _router_logits_body,
        grid=(2,),
        in_specs=[
            pl.BlockSpec((1, P2, DIM), lambda s: (s, 0, 0)),
            pl.BlockSpec((1, DIM, QK), lambda s: (s, 0, 0)),
            pl.BlockSpec((1, 1, QK), lambda s: (s, 0, 0)),
            pl.BlockSpec((1, DIM, QK), lambda s: (s, 0, 0)),
            pl.BlockSpec((1, 1, QK), lambda s: (s, 0, 0)),
        ],
        out_specs=pl.BlockSpec((1, P2, 64), lambda s: (s, 0, 0)),
        out_shape=jax.ShapeDtypeStruct((2, P2, 64), jnp.float32),
    )(xm2, Wq, bq, Wk, bk)


def _topk98(logits2):
    info = plsc.get_sparse_core_info()
    nc, ns = info.num_cores, info.num_subcores
    workers = nc * ns
    rpw = (ROWS + workers - 1) // workers
    mesh = plsc.VectorSubcoreMesh(core_axis_name="c", subcore_axis_name="s")

    @functools.partial(
        pl.kernel,
        mesh=mesh,
        compiler_params=pltpu.CompilerParams(needs_layout_passes=False),
        out_type=jax.ShapeDtypeStruct((ROWS * 8,), jnp.int32),
        scratch_types=[
            pltpu.VMEM((64,), jnp.float32),
            pltpu.VMEM((64,), jnp.float32),
            pltpu.VMEM((64,), jnp.int32),
            pltpu.VMEM((16,), jnp.int32),
        ],
    )
    def run(lg_hbm, out_hbm, lrow, skeys, svals, fvals):
        wid = lax.axis_index("s") * nc + lax.axis_index("c")
        for r in range(rpw):
            row = wid * rpw + r

            @pl.when(row < ROWS)
            def _():
                pltpu.sync_copy(lg_hbm.at[pl.ds(row * 64, 64)], lrow)
                for c in range(4):
                    keys = -lrow[pl.ds(16 * c, 16)]
                    vals = lax.iota(jnp.int32, 16) + 16 * c
                    sk, sv = plsc.sort_key_val(keys, vals)
                    skeys[pl.ds(16 * c, 16)] = sk
                    svals[pl.ds(16 * c, 16)] = sv
                lane = lax.iota(jnp.int32, 16)
                cand = (lane // 4) * 16 + (lane % 4)
                ck = plsc.load_gather(skeys, [cand])
                cv = plsc.load_gather(svals, [cand])
                _, fv = plsc.sort_key_val(ck, cv)
                fvals[...] = fv
                pltpu.sync_copy(fvals.at[pl.ds(0, 8)],
                                out_hbm.at[pl.ds(row * 8, 8)])

    return run(logits2)


def _attn_body(idx_ref, x_ref, wq_ref, bq_ref, kvp_ref, o_ref):
    s = pl.program_id(0)
    w = pl.program_id(1)
    xw = x_ref[0].reshape(WS * WS, DIM)
    q = jnp.dot(xw, wq_ref[0], preferred_element_type=jnp.float32, precision=lax.Precision.HIGHEST) + bq_ref[0]
    kv = jnp.concatenate(
        [kvp_ref[0, idx_ref[s, w, t]] for t in range(TOPK)], axis=0)
    k = kv[:, :QK]
    v = kv[:, QK:]
    col = lax.broadcasted_iota(jnp.int32, (1, QK), 1)
    o = jnp.zeros((WS * WS, DIM), jnp.float32)
    for h in range(HEADS):
        m = (col >= HD * h) & (col < HD * (h + 1))
        kh = jnp.where(m, k, 0.0)
        sh = lax.dot_general(q, kh, (((1,), (1,)), ((), ())),
                             preferred_element_type=jnp.float32, precision=lax.Precision.HIGHEST) * SCALE
        sh = sh - jnp.max(sh, axis=1, keepdims=True)
        p = jnp.exp(sh)
        p = p / jnp.sum(p, axis=1, keepdims=True)
        vh = jnp.where(m, v, 0.0)
        o = o + jnp.dot(p, vh, preferred_element_type=jnp.float32, precision=lax.Precision.HIGHEST)
    o_ref[0] = o.reshape(WS, WS, DIM)


def _run_attention(idx, x2, Wq, bq, kvp):
    grid_spec = pltpu.PrefetchScalarGridSpec(
        num_scalar_prefetch=1,
        grid=(2, P2),
        in_specs=[
            pl.BlockSpec((1, WS, WS, DIM),
                         lambda s, w, idx: (s, w // NWIN, w % NWIN, 0)),
            pl.BlockSpec((1, DIM, QK), lambda s, w, idx: (s, 0, 0)),
            pl.BlockSpec((1, 1, QK), lambda s, w, idx: (s, 0, 0)),
            pl.BlockSpec((1, P2, W2K, CKV), lambda s, w, idx: (s, 0, 0, 0)),
        ],
        out_specs=pl.BlockSpec((1, WS, WS, DIM),
                               lambda s, w, idx: (s, w // NWIN, w % NWIN, 0)),
    )
    return pl.pallas_call(
        _attn_body,
        grid_spec=grid_spec,
        out_shape=jax.ShapeDtypeStruct((2, 224, 224, DIM), jnp.float32),
    )(idx, x2, Wq, bq, kvp)


def _lepe_proj_body(attn_ref, vp_ref, vc_ref, vn_ref, w9_ref, bl_ref,
                    wo_ref, bo_ref, o_ref):
    j = pl.program_id(1)
    top = jnp.where(j == 0, 0.0, vp_ref[0, WS - 1:WS])
    bot = jnp.where(j == NWIN - 1, 0.0, vn_ref[0, 0:1])
    ext = jnp.concatenate([top, vc_ref[0], bot], axis=0)
    acc = jnp.zeros((WS, 224, DIM), jnp.float32)
    zcol = jnp.zeros((WS, 1, DIM), jnp.float32)
    for a in range(3):
        rows = ext[a:a + WS]
        for b in range(3):
            if b == 0:
                sh = jnp.concatenate([zcol, rows[:, :223]], axis=1)
            elif b == 2:
                sh = jnp.concatenate([rows[:, 1:], zcol], axis=1)
            else:
                sh = rows
            acc = acc + sh * w9_ref[0, 3 * a + b]
    tot = attn_ref[0] + acc + bl_ref[0, 0]
    flat = tot.reshape(WS * 224, DIM)
    out = jnp.dot(flat, wo_ref[...], preferred_element_type=jnp.float32, precision=lax.Precision.HIGHEST) + bo_ref[...]
    o_ref[0] = out.reshape(WS, 224, DIM)


def _run_lepe_proj(attn, vimg, w9, bl, W_wo, b_wo):
    return pl.pallas_call(
        _lepe_proj_body,
        grid=(2, NWIN),
        in_specs=[
            pl.BlockSpec((1, WS, 224, DIM), lambda s, j: (s, j, 0, 0)),
            pl.BlockSpec((1, WS, 224, DIM),
                         lambda s, j: (s, jnp.maximum(j - 1, 0), 0, 0)),
            pl.BlockSpec((1, WS, 224, DIM), lambda s, j: (s, j, 0, 0)),
            pl.BlockSpec((1, WS, 224, DIM),
                         lambda s, j: (s, jnp.minimum(j + 1, NWIN - 1), 0, 0)),
            pl.BlockSpec((1, 9, DIM), lambda s, j: (s, 0, 0)),
            pl.BlockSpec((1, 1, DIM), lambda s, j: (s, 0, 0)),
            pl.BlockSpec((DIM, DIM), lambda s, j: (0, 0)),
            pl.BlockSpec((1, DIM), lambda s, j: (0, 0)),
        ],
        out_specs=pl.BlockSpec((1, WS, 224, DIM), lambda s, j: (s, j, 0, 0)),
        out_shape=jax.ShapeDtypeStruct((2, 224, 224, DIM), jnp.float32),
    )(attn, vimg, vimg, vimg, w9, bl, W_wo, b_wo)


def kernel(x, y, W_qkv_x, b_qkv_x, W_qkv_y, b_qkv_y,
           lepe_x_w, lepe_x_b, lepe_y_w, lepe_y_b, W_wo, b_wo):
    x2 = jnp.concatenate([x, y], axis=0)
    Wq = jnp.stack([W_qkv_x[:, :QK], W_qkv_y[:, :QK]])
    Wk = jnp.stack([W_qkv_x[:, QK:2 * QK], W_qkv_y[:, QK:2 * QK]])
    Wv = jnp.stack([W_qkv_x[:, 2 * QK:], W_qkv_y[:, 2 * QK:]])
    Wkv = jnp.stack([W_qkv_x[:, QK:], W_qkv_y[:, QK:]])
    bq = jnp.stack([b_qkv_x[:QK], b_qkv_y[:QK]]).reshape(2, 1, QK)
    bk = jnp.stack([b_qkv_x[QK:2 * QK], b_qkv_y[QK:2 * QK]]).reshape(2, 1, QK)
    bv = jnp.stack([b_qkv_x[2 * QK:], b_qkv_y[2 * QK:]]).reshape(2, 1, DIM)
    bkv = jnp.stack([b_qkv_x[QK:], b_qkv_y[QK:]]).reshape(2, 1, CKV)
    w9 = jnp.stack([lepe_x_w.reshape(DIM, 9).T, lepe_y_w.reshape(DIM, 9).T])
    bl = jnp.stack([lepe_x_b, lepe_y_b]).reshape(2, 1, DIM)

    vimg, kvp, xm = _run_proj_pool(x2, Wv, bv, Wkv, bkv)
    logits = _run_router_logits(xm.reshape(2, P2, DIM), Wq, bq, Wk, bk)
    idx = _topk98(logits.reshape(ROWS * 64))
    attn = _run_attention(idx.reshape(2, P2, 8), x2, Wq, bq, kvp)
    out = _run_lepe_proj(attn, vimg, w9, bl, W_wo, b_wo.reshape(1, DIM))
    return (out[0:1], out[1:2])


# speedup vs baseline: 1.4661x; 1.2140x over previous
import functools

import jax
import jax.numpy as jnp
from jax import lax
from jax.experimental import pallas as pl
from jax.experimental.pallas import tpu as pltpu
from jax.experimental.pallas import tpu_sc as plsc

DIM = 96
QK = 96
NWIN = 7
HEADS = 8
TOPK = 4
RATIO = 4
SCALE = QK ** -0.5
WS = 32
P2 = NWIN * NWIN
W2K = (WS // RATIO) ** 2
HD = QK // HEADS
CKV = QK + DIM
SROW = 56
ROWS = 2 * SROW


def _proj_pool_body(x_ref, wv_ref, bv_ref, wkv_ref, bkv_ref,
                    v_ref, kvp_ref, xm_ref):
    xw = x_ref[...].reshape(WS * WS, DIM)
    v = jnp.dot(xw, wv_ref[...], preferred_element_type=jnp.float32,
                precision=lax.Precision.HIGHEST) + bv_ref[...]
    v_ref[...] = v.reshape(WS, WS, DIM)
    t = x_ref[...].reshape(WS, WS // RATIO, RATIO, DIM)
    t = t[:, :, 0] + t[:, :, 1] + t[:, :, 2] + t[:, :, 3]
    t = t.reshape(WS // RATIO, RATIO, WS // RATIO, DIM)
    t = t[:, 0] + t[:, 1] + t[:, 2] + t[:, 3]
    pooled = t.reshape(W2K, DIM) * (1.0 / (RATIO * RATIO))
    kvp_ref[0] = jnp.dot(pooled, wkv_ref[...],
                         preferred_element_type=jnp.float32,
                         precision=lax.Precision.HIGHEST) + bkv_ref[...]
    xm_ref[0] = jnp.mean(xw, axis=0).reshape(1, DIM)


def _run_proj_pool(img, Wv, bv, Wkv, bkv):
    return pl.pallas_call(
        _proj_pool_body,
        grid=(NWIN, NWIN),
        in_specs=[
            pl.BlockSpec((WS, WS, DIM), lambda j, i: (j, i, 0)),
            pl.BlockSpec((DIM, DIM), lambda j, i: (0, 0)),
            pl.BlockSpec((1, DIM), lambda j, i: (0, 0)),
            pl.BlockSpec((DIM, CKV), lambda j, i: (0, 0)),
            pl.BlockSpec((1, CKV), lambda j, i: (0, 0)),
        ],
        out_specs=[
            pl.BlockSpec((WS, WS, DIM), lambda j, i: (j, i, 0)),
            pl.BlockSpec((1, W2K, CKV), lambda j, i: (j * NWIN + i, 0, 0)),
            pl.BlockSpec((1, 1, DIM), lambda j, i: (j * NWIN + i, 0, 0)),
        ],
        out_shape=[
            jax.ShapeDtypeStruct((224, 224, DIM), jnp.float32),
            jax.ShapeDtypeStruct((P2, W2K, CKV), jnp.float32),
            jax.ShapeDtypeStruct((P2, 1, DIM), jnp.float32),
        ],
    )(img, Wv, bv, Wkv, bkv)


def _router_logits_body(xmx_ref, xmy_ref, wq_ref, bq_ref, wk_ref, bk_ref,
                        out_ref):
    col = lax.broadcasted_iota(jnp.int32, (SROW, 128), 1)
    for s, xm_ref in enumerate((xmx_ref, xmy_ref)):
        xm = xm_ref[...].reshape(P2, DIM)
        qw = jnp.dot(xm, wq_ref[s], preferred_element_type=jnp.float32,
                     precision=lax.Precision.HIGHEST) + bq_ref[s]
        kw = jnp.dot(xm, wk_ref[s], preferred_element_type=jnp.float32,
                     precision=lax.Precision.HIGHEST) + bk_ref[s]
        qw_pad = jnp.concatenate(
            [qw, jnp.zeros((SROW - P2, QK), jnp.float32)], axis=0)
        kw_pad = jnp.concatenate(
            [kw, jnp.zeros((128 - P2, QK), jnp.float32)], axis=0)
        lg = lax.dot_general(qw_pad * SCALE, kw_pad, (((1,), (1,)), ((), ())),
                             preferred_element_type=jnp.float32,
                             precision=lax.Precision.HIGHEST)
        out_ref[pl.ds(s * SROW, SROW), :] = jnp.where(col < P2, lg, -1e30)


def _run_router_logits(xmx, xmy, Wq, bq, Wk, bk):
    return pl.pallas_call(
        _router_logits_body,
        out_shape=jax.ShapeDtypeStruct((ROWS, 128), jnp.float32),
    )(xmx, xmy, Wq, bq, Wk, bk)


def _topk98(logits2):
    info = plsc.get_sparse_core_info()
    nc, ns = info.num_cores, info.num_subcores
    workers = nc * ns
    rpw = (ROWS + workers - 1) // workers
    mesh = plsc.VectorSubcoreMesh(core_axis_name="c", subcore_axis_name="s")

    @functools.partial(
        pl.kernel,
        mesh=mesh,
        compiler_params=pltpu.CompilerParams(needs_layout_passes=False),
        out_type=jax.ShapeDtypeStruct((ROWS * 8,), jnp.int32),
        scratch_types=[
            pltpu.VMEM((16,), jnp.int32),
            pltpu.VMEM((16, 128), jnp.float32),
            pltpu.VMEM((64,), jnp.float32),
            pltpu.VMEM((64,), jnp.int32),
            pltpu.VMEM((16,), jnp.int32),
            pltpu.SemaphoreType.DMA,
        ],
    )
    def run(lg_hbm, out_hbm, idx_v, rows_v, skeys, svals, fvals, sem):
        wid = lax.axis_index("s") * nc + lax.axis_index("c")
        lane = lax.iota(jnp.int32, 16)
        idx_v[...] = jnp.minimum(wid * rpw + lane, ROWS - 1)
        pltpu.async_copy(lg_hbm.at[idx_v], rows_v, sem).wait()
        for r in range(rpw):
            row = wid * rpw + r

            @pl.when(row < ROWS)
            def _():
                for c in range(4):
                    keys = -rows_v[r, pl.ds(16 * c, 16)]
                    vals = lane + 16 * c
                    sk, sv = plsc.sort_key_val(keys, vals)
                    skeys[pl.ds(16 * c, 16)] = sk
                    svals[pl.ds(16 * c, 16)] = sv
                cand = (lane // 4) * 16 + (lane % 4)
                ck = plsc.load_gather(skeys, [cand])
                cv = plsc.load_gather(svals, [cand])
                _, fv = plsc.sort_key_val(ck, cv)
                fvals[...] = fv
                pltpu.sync_copy(fvals.at[pl.ds(0, 8)],
                                out_hbm.at[pl.ds(row * 8, 8)])

    return run(logits2)


def _make_attn_body(stream):
    def _attn_body(idx_ref, x_ref, wq_ref, bq_ref, kvp_ref, o_ref):
        w = pl.program_id(0)
        xw = x_ref[...].reshape(WS * WS, DIM)
        q = jnp.dot(xw, wq_ref[...],
                    preferred_element_type=jnp.float32) + bq_ref[...]
        base = (stream * SROW + w) * 8
        kv = jnp.concatenate(
            [kvp_ref[idx_ref[base + t]] for t in range(TOPK)], axis=0)
        k = kv[:, :QK]
        v = kv[:, QK:]
        col = lax.broadcasted_iota(jnp.int32, (1, QK), 1)
        o = jnp.zeros((WS * WS, DIM), jnp.float32)
        for h in range(HEADS):
            m = (col >= HD * h) & (col < HD * (h + 1))
            kh = jnp.where(m, k, 0.0)
            sh = lax.dot_general(q, kh, (((1,), (1,)), ((), ())),
                                 preferred_element_type=jnp.float32) * SCALE
            sh = sh - jnp.max(sh, axis=1, keepdims=True)
            p = jnp.exp(sh)
            p = p / jnp.sum(p, axis=1, keepdims=True)
            vh = jnp.where(m, v, 0.0)
            o = o + jnp.dot(p, vh, preferred_element_type=jnp.float32,
                            precision=lax.Precision.HIGHEST)
        o_ref[...] = o.reshape(WS, WS, DIM)
    return _attn_body


def _run_attention(stream, idx, img, Wq, bq, kvp):
    grid_spec = pltpu.PrefetchScalarGridSpec(
        num_scalar_prefetch=1,
        grid=(P2,),
        in_specs=[
            pl.BlockSpec((WS, WS, DIM),
                         lambda w, idx: (w // NWIN, w % NWIN, 0)),
            pl.BlockSpec((DIM, QK), lambda w, idx: (0, 0)),
            pl.BlockSpec((1, QK), lambda w, idx: (0, 0)),
            pl.BlockSpec((P2, W2K, CKV), lambda w, idx: (0, 0, 0)),
        ],
        out_specs=pl.BlockSpec((WS, WS, DIM),
                               lambda w, idx: (w // NWIN, w % NWIN, 0)),
    )
    return pl.pallas_call(
        _make_attn_body(stream),
        grid_spec=grid_spec,
        out_shape=jax.ShapeDtypeStruct((224, 224, DIM), jnp.float32),
    )(idx, img, Wq, bq, kvp)


def _lepe_proj_body(attn_ref, vp_ref, vc_ref, vn_ref, w9_ref, bl_ref,
                    wo_ref, bo_ref, o_ref):
    j = pl.program_id(0)
    top = jnp.where(j == 0, 0.0, vp_ref[WS - 1:WS])
    bot = jnp.where(j == NWIN - 1, 0.0, vn_ref[0:1])
    ext = jnp.concatenate([top, vc_ref[...], bot], axis=0)
    acc = jnp.zeros((WS, 224, DIM), jnp.float32)
    zcol = jnp.zeros((WS, 1, DIM), jnp.float32)
    for a in range(3):
        rows = ext[a:a + WS]
        for b in range(3):
            if b == 0:
                sh = jnp.concatenate([zcol, rows[:, :223]], axis=1)
            elif b == 2:
                sh = jnp.concatenate([rows[:, 1:], zcol], axis=1)
            else:
                sh = rows
            acc = acc + sh * w9_ref[3 * a + b]
    tot = attn_ref[...] + acc + bl_ref[...]
    flat = tot.reshape(WS * 224, DIM)
    out = jnp.dot(flat, wo_ref[...], preferred_element_type=jnp.float32,
                  precision=lax.Precision.HIGHEST) + bo_ref[...]
    o_ref[...] = out.reshape(WS, 224, DIM)


def _run_lepe_proj(attn, vimg, w9, bl, W_wo, b_wo):
    return pl.pallas_call(
        _lepe_proj_body,
        grid=(NWIN,),
        in_specs=[
            pl.BlockSpec((WS, 224, DIM), lambda j: (j, 0, 0)),
            pl.BlockSpec((WS, 224, DIM), lambda j: (jnp.maximum(j - 1, 0), 0, 0)),
            pl.BlockSpec((WS, 224, DIM), lambda j: (j, 0, 0)),
            pl.BlockSpec((WS, 224, DIM),
                         lambda j: (jnp.minimum(j + 1, NWIN - 1), 0, 0)),
            pl.BlockSpec((9, DIM), lambda j: (0, 0)),
            pl.BlockSpec((1, DIM), lambda j: (0, 0)),
            pl.BlockSpec((DIM, DIM), lambda j: (0, 0)),
            pl.BlockSpec((1, DIM), lambda j: (0, 0)),
        ],
        out_specs=pl.BlockSpec((WS, 224, DIM), lambda j: (j, 0, 0)),
        out_shape=jax.ShapeDtypeStruct((224, 224, DIM), jnp.float32),
    )(attn, vimg, vimg, vimg, w9, bl, W_wo, b_wo)


def kernel(x, y, W_qkv_x, b_qkv_x, W_qkv_y, b_qkv_y,
           lepe_x_w, lepe_x_b, lepe_y_w, lepe_y_b, W_wo, b_wo):
    Wq = jnp.stack([W_qkv_x[:, :QK], W_qkv_y[:, :QK]])
    Wk = jnp.stack([W_qkv_x[:, QK:2 * QK], W_qkv_y[:, QK:2 * QK]])
    bq = jnp.stack([b_qkv_x[:QK], b_qkv_y[:QK]]).reshape(2, 1, QK)
    bk = jnp.stack([b_qkv_x[QK:2 * QK], b_qkv_y[QK:2 * QK]]).reshape(2, 1, QK)
    bwo = b_wo.reshape(1, DIM)

    outs = []
    per_stream = []
    for s, (img, Wf, bf, lw, lb) in enumerate((
            (x[0], W_qkv_x, b_qkv_x, lepe_x_w, lepe_x_b),
            (y[0], W_qkv_y, b_qkv_y, lepe_y_w, lepe_y_b))):
        Wv = Wf[:, 2 * QK:]
        bv = bf[2 * QK:].reshape(1, DIM)
        Wkv = Wf[:, QK:]
        bkv = bf[QK:].reshape(1, CKV)
        vimg, kvp, xm = _run_proj_pool(img, Wv, bv, Wkv, bkv)
        w9 = lw.reshape(DIM, 9).T
        per_stream.append((img, vimg, kvp, xm, w9, lb.reshape(1, DIM)))

    logits = _run_router_logits(per_stream[0][3], per_stream[1][3],
                                Wq, bq, Wk, bk)
    idx = _topk98(logits)

    for s, (img, vimg, kvp, xm, w9, bl) in enumerate(per_stream):
        attn = _run_attention(s, idx, img, Wq[s], bq[s], kvp)
        out = _run_lepe_proj(attn, vimg, w9, bl, W_wo, bwo)
        outs.append(out[None])
    return (outs[0], outs[1])
